# Initial kernel scaffold; baseline (speedup 1.0000x reference)
import jax
import jax.numpy as jnp
from jax.experimental import pallas as pl


def kernel(x, edge_index, W, b):
    raise NotImplementedError("write your pallas kernel here")


# speedup vs baseline: 8.9144x; 8.9144x over previous
import functools
import jax
import jax.numpy as jnp
from jax import lax
from jax.experimental import pallas as pl
from jax.experimental.pallas import tpu as pltpu, tpu_sc as plsc

N_NODES = 10000
N_EDGES = 320000
D = 128

NC = 2
NS = 16
NW = NC * NS

NP = 10240
EP = NW * NP
CHUNK = 128
CHUNKS_PER_TILE = EP // NW // CHUNK
ROWS_PER_TILE = NP // NS

_mesh = plsc.VectorSubcoreMesh(core_axis_name="c", subcore_axis_name="s")


def _deg_body(src_hbm, zeros_hbm, cnt_hbm, idx_v, ones_v, acc_sh, sem):
    cid = lax.axis_index("c")
    sid = lax.axis_index("s")
    wid = cid * NS + sid
    pltpu.sync_copy(zeros_hbm, acc_sh.at[pl.ds(sid * ROWS_PER_TILE, ROWS_PER_TILE)])
    for i in range(CHUNK // 16):
        ones_v[pl.ds(i * 16, 16)] = jnp.full((16,), 1.0, jnp.float32)
    plsc.subcore_barrier()

    ebase = wid * (EP // NW)

    def step(c, carry):
        base = pl.multiple_of(ebase + c * CHUNK, 8)
        pltpu.sync_copy(src_hbm.at[pl.ds(base, CHUNK)], idx_v)
        pltpu.sync_copy(ones_v, acc_sh.at[idx_v], add=True)
        return carry

    lax.fori_loop(0, CHUNKS_PER_TILE, step, 0)
    plsc.subcore_barrier()
    pltpu.sync_copy(acc_sh.at[pl.ds(sid * ROWS_PER_TILE, ROWS_PER_TILE)],
                    cnt_hbm.at[cid, pl.ds(sid * ROWS_PER_TILE, ROWS_PER_TILE)])


_deg_kernel = functools.partial(
    pl.kernel,
    out_type=jax.ShapeDtypeStruct((NC, NP), jnp.float32),
    mesh=_mesh,
    scratch_types=[
        pltpu.VMEM((CHUNK,), jnp.int32),
        pltpu.VMEM((CHUNK,), jnp.float32),
        pltpu.VMEM_SHARED((NP,), jnp.float32),
        pltpu.SemaphoreType.DMA,
    ],
)(_deg_body)


def _edge_body(src_hbm, tar_hbm, featg_hbm, zeros_hbm, part_hbm,
               idx_s, idx_t, rows_v, acc_sh, sem):
    cid = lax.axis_index("c")
    sid = lax.axis_index("s")
    wid = cid * NS + sid
    pltpu.sync_copy(zeros_hbm, acc_sh.at[pl.ds(sid * ROWS_PER_TILE, ROWS_PER_TILE)])
    plsc.subcore_barrier()

    ebase = wid * (EP // NW)

    def step(c, carry):
        base = pl.multiple_of(ebase + c * CHUNK, 8)
        pltpu.sync_copy(tar_hbm.at[pl.ds(base, CHUNK)], idx_t)
        pltpu.sync_copy(src_hbm.at[pl.ds(base, CHUNK)], idx_s)
        pltpu.async_copy(featg_hbm.at[idx_t], rows_v, sem).wait()
        pltpu.sync_copy(rows_v, acc_sh.at[idx_s], add=True)
        return carry

    lax.fori_loop(0, CHUNKS_PER_TILE, step, 0)
    plsc.subcore_barrier()
    pltpu.sync_copy(acc_sh.at[pl.ds(sid * ROWS_PER_TILE, ROWS_PER_TILE)],
                    part_hbm.at[cid, pl.ds(sid * ROWS_PER_TILE, ROWS_PER_TILE)])


_edge_kernel = functools.partial(
    pl.kernel,
    out_type=jax.ShapeDtypeStruct((NC, NP, D), jnp.float32),
    mesh=_mesh,
    scratch_types=[
        pltpu.VMEM((CHUNK,), jnp.int32),
        pltpu.VMEM((CHUNK,), jnp.int32),
        pltpu.VMEM((CHUNK, D), jnp.float32),
        pltpu.VMEM_SHARED((NP, D), jnp.float32),
        pltpu.SemaphoreType.DMA,
    ],
)(_edge_body)


ROW_BLK = 2048


def _dense_body(x_ref, wt_ref, b_ref, c0_ref, c1_ref,
                featg_ref, selfterm_ref, g_ref):
    feat = jnp.dot(x_ref[...], wt_ref[...],
                   preferred_element_type=jnp.float32) + b_ref[...]
    deg = c0_ref[...] + c1_ref[...] + 1.0
    g = lax.rsqrt(deg)
    featg_ref[...] = feat * g
    selfterm_ref[...] = feat / deg
    g_ref[...] = g


def _dense_call(xp, wt, b2, c0, c1):
    grid = NP // ROW_BLK
    return pl.pallas_call(
        _dense_body,
        grid=(grid,),
        in_specs=[
            pl.BlockSpec((ROW_BLK, D), lambda i: (i, 0)),
            pl.BlockSpec((D, D), lambda i: (0, 0)),
            pl.BlockSpec((1, D), lambda i: (0, 0)),
            pl.BlockSpec((ROW_BLK, 1), lambda i: (i, 0)),
            pl.BlockSpec((ROW_BLK, 1), lambda i: (i, 0)),
        ],
        out_specs=[
            pl.BlockSpec((ROW_BLK, D), lambda i: (i, 0)),
            pl.BlockSpec((ROW_BLK, D), lambda i: (i, 0)),
            pl.BlockSpec((ROW_BLK, 1), lambda i: (i, 0)),
        ],
        out_shape=[
            jax.ShapeDtypeStruct((NP, D), jnp.float32),
            jax.ShapeDtypeStruct((NP, D), jnp.float32),
            jax.ShapeDtypeStruct((NP, 1), jnp.float32),
        ],
    )(xp, wt, b2, c0, c1)


def _combine_body(p0_ref, p1_ref, g_ref, selfterm_ref, out_ref):
    out_ref[...] = g_ref[...] * (p0_ref[...] + p1_ref[...]) + selfterm_ref[...]


def _combine_call(p0, p1, g, selfterm):
    grid = NP // ROW_BLK
    return pl.pallas_call(
        _combine_body,
        grid=(grid,),
        in_specs=[
            pl.BlockSpec((ROW_BLK, D), lambda i: (i, 0)),
            pl.BlockSpec((ROW_BLK, D), lambda i: (i, 0)),
            pl.BlockSpec((ROW_BLK, 1), lambda i: (i, 0)),
            pl.BlockSpec((ROW_BLK, D), lambda i: (i, 0)),
        ],
        out_specs=pl.BlockSpec((ROW_BLK, D), lambda i: (i, 0)),
        out_shape=jax.ShapeDtypeStruct((NP, D), jnp.float32),
    )(p0, p1, g, selfterm)


def kernel(x, edge_index, W, b):
    src = edge_index[0]
    tar = edge_index[1]
    pad_e = EP - N_EDGES
    srcp = jnp.concatenate([src, jnp.full((pad_e,), NP - 1, jnp.int32)])
    tarp = jnp.concatenate([tar, jnp.zeros((pad_e,), jnp.int32)])

    zeros1 = jnp.zeros((ROWS_PER_TILE,), jnp.float32)
    counts = _deg_kernel(srcp, zeros1)

    xp = jnp.pad(x, ((0, NP - N_NODES), (0, 0)))
    c0 = counts[0].reshape(NP, 1)
    c1 = counts[1].reshape(NP, 1)
    featg, selfterm, g = _dense_call(xp, W.T, b.reshape(1, D), c0, c1)

    zeros2 = jnp.zeros((ROWS_PER_TILE, D), jnp.float32)
    parts = _edge_kernel(srcp, tarp, featg, zeros2)

    out = _combine_call(parts[0], parts[1], g, selfterm)
    return out[:N_NODES]


# speedup vs baseline: 12.6927x; 1.4238x over previous
import functools
import jax
import jax.numpy as jnp
from jax import lax
from jax.experimental import pallas as pl
from jax.experimental.pallas import tpu as pltpu, tpu_sc as plsc

N_NODES = 10000
N_EDGES = 320000
D = 128

NC = 2
NS = 16
NW = NC * NS

NP = 10240
EP = NW * NP
DCH = 128
DCHUNKS = EP // NW // DCH
CHUNK = 64
CHUNKS_PER_TILE = EP // NW // CHUNK
ROWS_PER_TILE = NP // NS

_mesh = plsc.VectorSubcoreMesh(core_axis_name="c", subcore_axis_name="s")


def _deg_body(src_hbm, zeros_hbm, cnt_hbm, idx_all, ones_v, acc_sh, sem):
    cid = lax.axis_index("c")
    sid = lax.axis_index("s")
    wid = cid * NS + sid
    pltpu.sync_copy(zeros_hbm, acc_sh.at[pl.ds(sid * ROWS_PER_TILE, ROWS_PER_TILE)])
    for i in range(DCH // 16):
        ones_v[pl.ds(i * 16, 16)] = jnp.full((16,), 1.0, jnp.float32)
    pltpu.sync_copy(src_hbm.at[wid], idx_all)
    plsc.subcore_barrier()

    def step(c, carry):
        pltpu.sync_copy(ones_v, acc_sh.at[idx_all.at[c]], add=True)
        return carry

    lax.fori_loop(0, DCHUNKS, step, 0)
    plsc.subcore_barrier()
    pltpu.sync_copy(acc_sh.at[pl.ds(sid * ROWS_PER_TILE, ROWS_PER_TILE)],
                    cnt_hbm.at[cid, pl.ds(sid * ROWS_PER_TILE, ROWS_PER_TILE)])


_deg_kernel = functools.partial(
    pl.kernel,
    out_type=jax.ShapeDtypeStruct((NC, NP), jnp.float32),
    mesh=_mesh,
    scratch_types=[
        pltpu.VMEM((DCHUNKS, DCH), jnp.int32),
        pltpu.VMEM((DCH,), jnp.float32),
        pltpu.VMEM_SHARED((NP,), jnp.float32),
        pltpu.SemaphoreType.DMA,
    ],
)(_deg_body)


NBUF = 4
NIB = 8


def _edge_body(src_hbm, tar_hbm, featg_hbm, zeros_hbm, part_hbm,
               idx_s, idx_t, rows_v, acc_sh, issems, itsems, gsems, ssems):
    cid = lax.axis_index("c")
    sid = lax.axis_index("s")
    wid = cid * NS + sid
    pltpu.sync_copy(zeros_hbm, acc_sh.at[pl.ds(sid * ROWS_PER_TILE, ROWS_PER_TILE)])
    plsc.subcore_barrier()

    sb = src_hbm.at[wid]
    tb = tar_hbm.at[wid]
    for k in range(NIB):
        pltpu.async_copy(sb.at[k], idx_s.at[k], issems[k])
        pltpu.async_copy(tb.at[k], idx_t.at[k], itsems[k])
    for k in range(NBUF):
        pltpu.make_async_copy(tb.at[k], idx_t.at[k], itsems[k]).wait()
        pltpu.async_copy(featg_hbm.at[idx_t.at[k]], rows_v.at[k], gsems[k])

    def round_step(r, carry):
        for k in range(NIB):
            c = r * NIB + k
            b = k % NBUF
            s4 = (k + NBUF) % NIB
            pltpu.make_async_copy(featg_hbm.at[idx_t.at[k]], rows_v.at[b],
                                  gsems[b]).wait()
            pltpu.make_async_copy(sb.at[k], idx_s.at[k], issems[k]).wait()
            pltpu.async_copy(rows_v.at[b], acc_sh.at[idx_s.at[k]],
                             ssems[b], add=True)
            pltpu.make_async_copy(rows_v.at[b], acc_sh.at[idx_s.at[k]],
                                  ssems[b]).wait()
            @pl.when(c + NBUF < CHUNKS_PER_TILE)
            def _():
                pltpu.make_async_copy(tb.at[s4], idx_t.at[s4],
                                      itsems[s4]).wait()
                pltpu.async_copy(featg_hbm.at[idx_t.at[s4]], rows_v.at[b],
                                 gsems[b])
            @pl.when(c + NIB < CHUNKS_PER_TILE)
            def _():
                pltpu.async_copy(sb.at[c + NIB], idx_s.at[k], issems[k])
                pltpu.async_copy(tb.at[c + NIB], idx_t.at[k], itsems[k])
        return carry

    lax.fori_loop(0, CHUNKS_PER_TILE // NIB, round_step, 0)
    plsc.subcore_barrier()
    pltpu.sync_copy(acc_sh.at[pl.ds(sid * ROWS_PER_TILE, ROWS_PER_TILE)],
                    part_hbm.at[cid, pl.ds(sid * ROWS_PER_TILE, ROWS_PER_TILE)])


_edge_kernel = functools.partial(
    pl.kernel,
    out_type=jax.ShapeDtypeStruct((NC, NP, D), jnp.float32),
    mesh=_mesh,
    scratch_types=[
        pltpu.VMEM((NIB, CHUNK), jnp.int32),
        pltpu.VMEM((NIB, CHUNK), jnp.int32),
        pltpu.VMEM((NBUF, CHUNK, D), jnp.float32),
        pltpu.VMEM_SHARED((NP, D), jnp.float32),
        [pltpu.SemaphoreType.DMA] * NIB,
        [pltpu.SemaphoreType.DMA] * NIB,
        [pltpu.SemaphoreType.DMA] * NBUF,
        [pltpu.SemaphoreType.DMA] * NBUF,
    ],
)(_edge_body)


ROW_BLK = 2048


def _matmul_body(x_ref, wt_ref, b_ref, feat_ref):
    feat_ref[...] = jnp.dot(x_ref[...], wt_ref[...],
                            preferred_element_type=jnp.float32) + b_ref[...]


def _matmul_call(xp, wt, b2):
    grid = NP // ROW_BLK
    return pl.pallas_call(
        _matmul_body,
        grid=(grid,),
        in_specs=[
            pl.BlockSpec((ROW_BLK, D), lambda i: (i, 0)),
            pl.BlockSpec((D, D), lambda i: (0, 0)),
            pl.BlockSpec((1, D), lambda i: (0, 0)),
        ],
        out_specs=pl.BlockSpec((ROW_BLK, D), lambda i: (i, 0)),
        out_shape=jax.ShapeDtypeStruct((NP, D), jnp.float32),
    )(xp, wt, b2)


def _scale_body(feat_ref, c0_ref, c1_ref, featg_ref, selfterm_ref, g_ref):
    feat = feat_ref[...]
    deg = c0_ref[...] + c1_ref[...] + 1.0
    g = lax.rsqrt(deg)
    featg_ref[...] = feat * g
    selfterm_ref[...] = feat / deg
    g_ref[...] = g


def _scale_call(feat, c0, c1):
    grid = NP // ROW_BLK
    return pl.pallas_call(
        _scale_body,
        grid=(grid,),
        in_specs=[
            pl.BlockSpec((ROW_BLK, D), lambda i: (i, 0)),
            pl.BlockSpec((ROW_BLK, 1), lambda i: (i, 0)),
            pl.BlockSpec((ROW_BLK, 1), lambda i: (i, 0)),
        ],
        out_specs=[
            pl.BlockSpec((ROW_BLK, D), lambda i: (i, 0)),
            pl.BlockSpec((ROW_BLK, D), lambda i: (i, 0)),
            pl.BlockSpec((ROW_BLK, 1), lambda i: (i, 0)),
        ],
        out_shape=[
            jax.ShapeDtypeStruct((NP, D), jnp.float32),
            jax.ShapeDtypeStruct((NP, D), jnp.float32),
            jax.ShapeDtypeStruct((NP, 1), jnp.float32),
        ],
    )(feat, c0, c1)


def _combine_body(p0_ref, p1_ref, g_ref, selfterm_ref, out_ref):
    out_ref[...] = (g_ref[...] * (p0_ref[...] + p1_ref[...])
                    + selfterm_ref[...])


def _combine_call(p0, p1, g, selfterm):
    grid = NP // ROW_BLK
    return pl.pallas_call(
        _combine_body,
        grid=(grid,),
        in_specs=[
            pl.BlockSpec((ROW_BLK, D), lambda i: (i, 0)),
            pl.BlockSpec((ROW_BLK, D), lambda i: (i, 0)),
            pl.BlockSpec((ROW_BLK, 1), lambda i: (i, 0)),
            pl.BlockSpec((ROW_BLK, D), lambda i: (i, 0)),
        ],
        out_specs=pl.BlockSpec((ROW_BLK, D), lambda i: (i, 0)),
        out_shape=jax.ShapeDtypeStruct((NP, D), jnp.float32),
    )(p0, p1, g, selfterm)


def kernel(x, edge_index, W, b):
    src = edge_index[0]
    tar = edge_index[1]
    pad_e = EP - N_EDGES
    srcp = jnp.concatenate([src, jnp.full((pad_e,), NP - 1, jnp.int32)])
    tarp = jnp.concatenate([tar, jnp.zeros((pad_e,), jnp.int32)])

    zeros1 = jnp.zeros((ROWS_PER_TILE,), jnp.float32)
    counts = _deg_kernel(srcp.reshape(NW, DCHUNKS, DCH), zeros1)

    xp = jnp.pad(x, ((0, NP - N_NODES), (0, 0)))
    feat = _matmul_call(xp, W.T, b.reshape(1, D))
    c0 = counts[0].reshape(NP, 1)
    c1 = counts[1].reshape(NP, 1)
    featg, selfterm, g = _scale_call(feat, c0, c1)

    zeros2 = jnp.zeros((ROWS_PER_TILE, D), jnp.float32)
    parts = _edge_kernel(srcp.reshape(NW, CHUNKS_PER_TILE, CHUNK),
                         tarp.reshape(NW, CHUNKS_PER_TILE, CHUNK),
                         featg, zeros2)

    out = _combine_call(parts[0], parts[1], g, selfterm)
    return out[:N_NODES]
